# baseline (speedup 1.0000x reference)
import functools

import jax
import jax.numpy as jnp
from jax import lax
from jax.experimental import pallas as pl
from jax.experimental.pallas import tpu as pltpu
from jax.experimental.pallas import tpu_sc as plsc

NUM_FIELDS = 26
DIM = 16
LANES = 16
BCHUNK = 4096


def _make_lookup(batch: int, vocab: int, nw: int, nc: int):
    num_rows = NUM_FIELDS * DIM
    rows_per_tile = num_rows // nw
    n_chunks = batch // BCHUNK

    mesh = plsc.VectorSubcoreMesh(core_axis_name="c", subcore_axis_name="s")

    @functools.partial(
        pl.kernel,
        out_type=jax.ShapeDtypeStruct((num_rows, batch), jnp.float32),
        mesh=mesh,
        compiler_params=pltpu.CompilerParams(needs_layout_passes=False),
        scratch_types=[
            pltpu.VMEM((vocab,), jnp.float32),
            pltpu.VMEM((batch,), jnp.int32),
            pltpu.VMEM((BCHUNK,), jnp.float32),
            pltpu.VMEM((BCHUNK,), jnp.float32),
            pltpu.SemaphoreType.DMA,
            pltpu.SemaphoreType.DMA,
        ],
    )
    def lookup_kernel(xt_hbm, tab_hbm, out_hbm, rowbuf, idxbuf,
                      outbuf0, outbuf1, wsem, rsem):
        wid = lax.axis_index("s") * nc + lax.axis_index("c")
        c0 = wid * rows_per_tile
        outbufs = [outbuf0, outbuf1]

        def drain_wb():
            pltpu.make_async_copy(out_hbm.at[0, pl.ds(0, BCHUNK)],
                                  outbuf0, wsem).wait()

        def row_body(r, fprev):
            c = c0 + r
            f = lax.shift_right_logical(c, 4)
            d = lax.bitwise_and(c, DIM - 1)
            rcp = pltpu.async_copy(tab_hbm.at[f, d], rowbuf, rsem)

            @pl.when(f != fprev)
            def _():
                pltpu.sync_copy(xt_hbm.at[f], idxbuf)

            @pl.when(r > 0)
            def _():
                drain_wb()
                drain_wb()
            rcp.wait()

            wbs = {}
            for k in range(n_chunks):
                buf = outbufs[k % 2]
                if k >= 2:
                    wbs[k - 2].wait()

                @plsc.parallel_loop(0, BCHUNK // LANES, unroll=16)
                def _(j):
                    vals = plsc.load_gather(
                        rowbuf, [idxbuf[pl.ds(k * BCHUNK + j * LANES, LANES)]])
                    buf[pl.ds(j * LANES, LANES)] = vals

                wbs[k] = pltpu.async_copy(
                    buf, out_hbm.at[c, pl.ds(k * BCHUNK, BCHUNK)], wsem)
            return f
        lax.fori_loop(0, rows_per_tile, row_body, jnp.int32(-1))
        drain_wb()
        drain_wb()

    return lookup_kernel


def kernel(x, tables):
    batch, num_fields = x.shape
    _, vocab, dim = tables.shape

    info = plsc.get_sparse_core_info()
    nw = info.num_cores * info.num_subcores

    assert dim == DIM and num_fields == NUM_FIELDS
    assert (num_fields * dim) % nw == 0 and batch % BCHUNK == 0

    xt = jnp.swapaxes(x.astype(jnp.int32), 0, 1)
    tab_t = jnp.swapaxes(tables, 1, 2)

    out_t = _make_lookup(batch, vocab, nw, info.num_cores)(xt, tab_t)
    return jnp.swapaxes(out_t, 0, 1).reshape(batch, num_fields * dim)


# speedup vs baseline: 1.0009x; 1.0009x over previous
import functools

import jax
import jax.numpy as jnp
from jax import lax
from jax.experimental import pallas as pl
from jax.experimental.pallas import tpu as pltpu
from jax.experimental.pallas import tpu_sc as plsc

NUM_FIELDS = 26
DIM = 16
LANES = 16
BCHUNK = 4096


def _make_lookup(batch: int, vocab: int, nw: int, nc: int):
    num_rows = NUM_FIELDS * DIM
    rows_per_tile = num_rows // nw
    n_chunks = batch // BCHUNK

    mesh = plsc.VectorSubcoreMesh(core_axis_name="c", subcore_axis_name="s")

    @functools.partial(
        pl.kernel,
        out_type=jax.ShapeDtypeStruct((num_rows, batch), jnp.float32),
        mesh=mesh,
        compiler_params=pltpu.CompilerParams(needs_layout_passes=False),
        scratch_types=[
            pltpu.VMEM((vocab,), jnp.float32),
            pltpu.VMEM((batch,), jnp.int32),
            pltpu.VMEM((BCHUNK,), jnp.float32),
            pltpu.VMEM((BCHUNK,), jnp.float32),
            pltpu.SemaphoreType.DMA,
        ],
    )
    def lookup_kernel(xt_hbm, tab_hbm, out_hbm, rowbuf, idxbuf,
                      outbuf0, outbuf1, wsem):
        wid = lax.axis_index("s") * nc + lax.axis_index("c")
        c0 = wid * rows_per_tile
        outbufs = [outbuf0, outbuf1]

        def drain_wb():
            pltpu.make_async_copy(out_hbm.at[0, pl.ds(0, BCHUNK)],
                                  outbuf0, wsem).wait()

        def row_body(r, fprev):
            c = c0 + r
            f = lax.shift_right_logical(c, 4)
            d = lax.bitwise_and(c, DIM - 1)
            pltpu.sync_copy(tab_hbm.at[f, d], rowbuf)

            @pl.when(f != fprev)
            def _():
                pltpu.sync_copy(xt_hbm.at[f], idxbuf)

            wbs = {}
            for k in range(n_chunks):
                buf = outbufs[k % 2]
                if k >= 2:
                    wbs[k - 2].wait()
                else:
                    @pl.when(r > 0)
                    def _():
                        drain_wb()

                @plsc.parallel_loop(0, BCHUNK // LANES, unroll=8)
                def _(j):
                    vals = plsc.load_gather(
                        rowbuf, [idxbuf[pl.ds(k * BCHUNK + j * LANES, LANES)]])
                    buf[pl.ds(j * LANES, LANES)] = vals

                wbs[k] = pltpu.async_copy(
                    buf, out_hbm.at[c, pl.ds(k * BCHUNK, BCHUNK)], wsem)
            return f
        lax.fori_loop(0, rows_per_tile, row_body, jnp.int32(-1))
        drain_wb()
        drain_wb()

    return lookup_kernel


def kernel(x, tables):
    batch, num_fields = x.shape
    _, vocab, dim = tables.shape

    info = plsc.get_sparse_core_info()
    nw = info.num_cores * info.num_subcores

    assert dim == DIM and num_fields == NUM_FIELDS
    assert (num_fields * dim) % nw == 0 and batch % BCHUNK == 0

    xt = jnp.swapaxes(x.astype(jnp.int32), 0, 1)
    tab_t = jnp.swapaxes(tables, 1, 2)

    out_t = _make_lookup(batch, vocab, nw, info.num_cores)(xt, tab_t)
    return jnp.swapaxes(out_t, 0, 1).reshape(batch, num_fields * dim)
